# Initial kernel scaffold; baseline (speedup 1.0000x reference)
import jax
import jax.numpy as jnp
from jax.experimental import pallas as pl


def kernel(feat, W1, b1, W2, b2, W3, b3, Cc, Cf, D1, db1, D2, db2, Wf, bf, Wr, br, Wb, bb, Wk, bk):
    raise NotImplementedError("write your pallas kernel here")


# speedup vs baseline: 1.7059x; 1.7059x over previous
import functools

import jax
import jax.numpy as jnp
from jax import lax
from jax.experimental import pallas as pl
from jax.experimental.pallas import tpu as pltpu

FEAT = 256
HID = 128
DM = 64
NC = 256
NF = 1024
ROLES = 8
B = 32768
BETA = 0.25

BB = 1024
NSTEPS = B // BB


_SQRT_HALF = 0.7071067811865476


def _gelu(x):
    return 0.5 * x * (1.0 + lax.erf(x * _SQRT_HALF))


def _argmin_onehot(dist, k):
    minv = jnp.min(dist, axis=1, keepdims=True)
    iota = lax.broadcasted_iota(jnp.int32, dist.shape, 1)
    idx = jnp.min(jnp.where(dist == minv, iota, k), axis=1, keepdims=True)
    onehot = (iota == idx).astype(jnp.float32)
    return idx, onehot


def _vq_kernel(
    feat_ref, W1_ref, b1_ref, W2_ref, b2_ref, W3_ref, b3_ref,
    Cc_ref, Cf_ref, D1_ref, db1_ref, D2_ref, db2_ref,
    Wf_ref, bf_ref, Wr_ref, br_ref, Wb_ref, bb_ref, Wk_ref, bk_ref,
    feat_out_ref, role_ref, bounce_ref, break_ref, z_ref, cidx_ref, fidx_ref,
    commit_c_ref, commit_f_ref, ent_c_ref, ent_f_ref, used_c_ref, used_f_ref,
    counts_c_acc, counts_f_acc, sse_acc,
):
    step = pl.program_id(0)

    @pl.when(step == 0)
    def _init():
        counts_c_acc[...] = jnp.zeros_like(counts_c_acc)
        counts_f_acc[...] = jnp.zeros_like(counts_f_acc)
        sse_acc[...] = jnp.zeros_like(sse_acc)

    dot = functools.partial(jnp.dot, preferred_element_type=jnp.float32)

    f = feat_ref[...]
    h = _gelu(dot(f, W1_ref[...]) + b1_ref[...])
    h = _gelu(dot(h, W2_ref[...]) + b2_ref[...])
    z = dot(h, W3_ref[...]) + b3_ref[...]
    z_ref[...] = z

    Cc = Cc_ref[...]
    ze = lax.dot_general(z, Cc, (((1,), (1,)), ((), ())),
                         preferred_element_type=jnp.float32)
    z2 = jnp.sum(z * z, axis=1, keepdims=True)
    e2 = jnp.sum(Cc * Cc, axis=1)[None, :]
    dist = z2 + e2 - 2.0 * ze
    cidx, onehot_c = _argmin_onehot(dist, NC)
    cq = dot(onehot_c, Cc)
    cidx_ref[...] = cidx
    counts_c_acc[...] += jnp.sum(onehot_c, axis=0, keepdims=True)
    sse_acc[:, 0:1] += jnp.sum((z - cq) ** 2).reshape(1, 1)

    res = z - cq
    Cf = Cf_ref[...]
    re = lax.dot_general(res, Cf, (((1,), (1,)), ((), ())),
                         preferred_element_type=jnp.float32)
    r2 = jnp.sum(res * res, axis=1, keepdims=True)
    f2 = jnp.sum(Cf * Cf, axis=1)[None, :]
    distf = r2 + f2 - 2.0 * re
    fidx, onehot_f = _argmin_onehot(distf, NF)
    fq = dot(onehot_f, Cf)
    fidx_ref[...] = fidx
    counts_f_acc[...] += jnp.sum(onehot_f, axis=0, keepdims=True)
    sse_acc[:, 1:2] += jnp.sum((res - fq) ** 2).reshape(1, 1)

    dec = cq + fq
    t = _gelu(dot(dec, D1_ref[...]) + db1_ref[...])
    t = _gelu(dot(t, D2_ref[...]) + db2_ref[...])
    feat_out_ref[...] = dot(t, Wf_ref[...]) + bf_ref[...]
    role_ref[...] = dot(t, Wr_ref[...]) + br_ref[...]
    bounce_ref[...] = dot(t, Wb_ref[...]) + bb_ref[...]
    break_ref[...] = dot(t, Wk_ref[...]) + bk_ref[...]

    @pl.when(step == NSTEPS - 1)
    def _finalize():
        inv = 1.0 / (B * DM)
        sse = sse_acc[...]
        commit_c_ref[...] = BETA * inv * sse[:, 0:1]
        commit_f_ref[...] = BETA * inv * sse[:, 1:2]
        cc = counts_c_acc[...]
        cf = counts_f_acc[...]
        uc = cc * (1.0 / B) + 1e-10
        uf = cf * (1.0 / B) + 1e-10
        ent_c_ref[...] = -jnp.sum(uc * jnp.log(uc)).reshape(1, 1)
        ent_f_ref[...] = -jnp.sum(uf * jnp.log(uf)).reshape(1, 1)
        used_c_ref[...] = jnp.sum((cc > 0).astype(jnp.int32)).reshape(1, 1)
        used_f_ref[...] = jnp.sum((cf > 0).astype(jnp.int32)).reshape(1, 1)


def _full(shape):
    nd = len(shape)
    return pl.BlockSpec(shape, lambda i: (0,) * nd)


def _batched(cols):
    return pl.BlockSpec((BB, cols), lambda i: (i, 0))


@jax.jit
def kernel(feat, W1, b1, W2, b2, W3, b3, Cc, Cf, D1, db1, D2, db2,
           Wf, bf, Wr, br, Wb, bb, Wk, bk):
    b1r, b2r, b3r = b1[None, :], b2[None, :], b3[None, :]
    db1r, db2r = db1[None, :], db2[None, :]
    bfr, brr, bbr, bkr = bf[None, :], br[None, :], bb[None, :], bk[None, :]

    out_shapes = (
        jax.ShapeDtypeStruct((B, FEAT), jnp.float32),
        jax.ShapeDtypeStruct((B, ROLES), jnp.float32),
        jax.ShapeDtypeStruct((B, 2), jnp.float32),
        jax.ShapeDtypeStruct((B, 2), jnp.float32),
        jax.ShapeDtypeStruct((B, DM), jnp.float32),
        jax.ShapeDtypeStruct((B, 1), jnp.int32),
        jax.ShapeDtypeStruct((B, 1), jnp.int32),
        jax.ShapeDtypeStruct((1, 1), jnp.float32),
        jax.ShapeDtypeStruct((1, 1), jnp.float32),
        jax.ShapeDtypeStruct((1, 1), jnp.float32),
        jax.ShapeDtypeStruct((1, 1), jnp.float32),
        jax.ShapeDtypeStruct((1, 1), jnp.int32),
        jax.ShapeDtypeStruct((1, 1), jnp.int32),
    )
    in_specs = [
        _batched(FEAT),
        _full((FEAT, HID)), _full((1, HID)),
        _full((HID, HID)), _full((1, HID)),
        _full((HID, DM)), _full((1, DM)),
        _full((NC, DM)), _full((NF, DM)),
        _full((DM, HID)), _full((1, HID)),
        _full((HID, HID)), _full((1, HID)),
        _full((HID, FEAT)), _full((1, FEAT)),
        _full((HID, ROLES)), _full((1, ROLES)),
        _full((HID, 2)), _full((1, 2)),
        _full((HID, 2)), _full((1, 2)),
    ]
    out_specs = (
        _batched(FEAT), _batched(ROLES), _batched(2), _batched(2),
        _batched(DM), _batched(1), _batched(1),
        _full((1, 1)), _full((1, 1)), _full((1, 1)), _full((1, 1)),
        _full((1, 1)), _full((1, 1)),
    )
    scratch = [
        pltpu.VMEM((1, NC), jnp.float32),
        pltpu.VMEM((1, NF), jnp.float32),
        pltpu.VMEM((1, 2), jnp.float32),
    ]

    outs = pl.pallas_call(
        _vq_kernel,
        grid=(NSTEPS,),
        in_specs=in_specs,
        out_specs=out_specs,
        out_shape=out_shapes,
        scratch_shapes=scratch,
    )(feat, W1, b1r, W2, b2r, W3, b3r, Cc, Cf, D1, db1r, D2, db2r,
      Wf, bfr, Wr, brr, Wb, bbr, Wk, bkr)

    (feat_out, role_logits, bounce_logits, break_logits, z, cidx, fidx,
     commit_c, commit_f, ent_c, ent_f, used_c, used_f) = outs

    return (feat_out, role_logits, bounce_logits, break_logits, z,
            cidx[:, 0], fidx[:, 0],
            commit_c[0, 0], commit_f[0, 0], ent_c[0, 0], ent_f[0, 0],
            used_c[0, 0], used_f[0, 0])


# speedup vs baseline: 1.7356x; 1.0174x over previous
import functools

import jax
import jax.numpy as jnp
from jax import lax
from jax.experimental import pallas as pl
from jax.experimental.pallas import tpu as pltpu

FEAT = 256
HID = 128
DM = 64
NC = 256
NF = 1024
ROLES = 8
B = 32768
BETA = 0.25

BB = 1024
NSTEPS = B // BB


_SQRT_HALF = 0.7071067811865476


def _gelu(x):
    return 0.5 * x * (1.0 + lax.erf(x * _SQRT_HALF))


def _nearest_onehot(x, cb):
    e2 = jnp.sum(cb * cb, axis=1)[None, :]
    xe = lax.dot_general(x, cb, (((1,), (1,)), ((), ())),
                         preferred_element_type=jnp.float32)
    score = xe - 0.5 * e2
    maxv = jnp.max(score, axis=1, keepdims=True)
    iota = lax.broadcasted_iota(jnp.int32, score.shape, 1)
    k = cb.shape[0]
    idx = jnp.min(jnp.where(score == maxv, iota, k), axis=1, keepdims=True)
    onehot = (iota == idx).astype(jnp.float32)
    return idx, onehot


def _vq_kernel(
    feat_ref, W1_ref, b1_ref, W2_ref, b2_ref, W3_ref, b3_ref,
    Cc_ref, Cf_ref, D1_ref, db1_ref, D2_ref, db2_ref,
    Wf_ref, bf_ref, Wr_ref, br_ref, Wb_ref, bb_ref, Wk_ref, bk_ref,
    feat_out_ref, role_ref, bounce_ref, break_ref, z_ref, cidx_ref, fidx_ref,
    commit_c_ref, commit_f_ref, ent_c_ref, ent_f_ref, used_c_ref, used_f_ref,
    counts_c_acc, counts_f_acc, sse_acc,
):
    step = pl.program_id(0)

    @pl.when(step == 0)
    def _init():
        counts_c_acc[...] = jnp.zeros_like(counts_c_acc)
        counts_f_acc[...] = jnp.zeros_like(counts_f_acc)
        sse_acc[...] = jnp.zeros_like(sse_acc)

    dot = functools.partial(jnp.dot, preferred_element_type=jnp.float32)

    f = feat_ref[...]
    h = _gelu(dot(f, W1_ref[...]) + b1_ref[...])
    h = _gelu(dot(h, W2_ref[...]) + b2_ref[...])
    z = dot(h, W3_ref[...]) + b3_ref[...]
    z_ref[...] = z

    Cc = Cc_ref[...]
    cidx, onehot_c = _nearest_onehot(z, Cc)
    cq = dot(onehot_c, Cc)
    cidx_ref[...] = cidx
    counts_c_acc[...] += jnp.sum(onehot_c, axis=0, keepdims=True)
    sse_acc[:, 0:1] += jnp.sum((z - cq) ** 2).reshape(1, 1)

    res = z - cq
    Cf = Cf_ref[...]
    fidx, onehot_f = _nearest_onehot(res, Cf)
    fq = dot(onehot_f, Cf)
    fidx_ref[...] = fidx
    counts_f_acc[...] += jnp.sum(onehot_f, axis=0, keepdims=True)
    sse_acc[:, 1:2] += jnp.sum((res - fq) ** 2).reshape(1, 1)

    dec = cq + fq
    t = _gelu(dot(dec, D1_ref[...]) + db1_ref[...])
    t = _gelu(dot(t, D2_ref[...]) + db2_ref[...])
    feat_out_ref[...] = dot(t, Wf_ref[...]) + bf_ref[...]
    role_ref[...] = dot(t, Wr_ref[...]) + br_ref[...]
    bounce_ref[...] = dot(t, Wb_ref[...]) + bb_ref[...]
    break_ref[...] = dot(t, Wk_ref[...]) + bk_ref[...]

    @pl.when(step == NSTEPS - 1)
    def _finalize():
        inv = 1.0 / (B * DM)
        sse = sse_acc[...]
        commit_c_ref[...] = BETA * inv * sse[:, 0:1]
        commit_f_ref[...] = BETA * inv * sse[:, 1:2]
        cc = counts_c_acc[...]
        cf = counts_f_acc[...]
        uc = cc * (1.0 / B) + 1e-10
        uf = cf * (1.0 / B) + 1e-10
        ent_c_ref[...] = -jnp.sum(uc * jnp.log(uc)).reshape(1, 1)
        ent_f_ref[...] = -jnp.sum(uf * jnp.log(uf)).reshape(1, 1)
        used_c_ref[...] = jnp.sum((cc > 0).astype(jnp.int32)).reshape(1, 1)
        used_f_ref[...] = jnp.sum((cf > 0).astype(jnp.int32)).reshape(1, 1)


def _full(shape):
    nd = len(shape)
    return pl.BlockSpec(shape, lambda i: (0,) * nd)


def _batched(cols):
    return pl.BlockSpec((BB, cols), lambda i: (i, 0))


@jax.jit
def kernel(feat, W1, b1, W2, b2, W3, b3, Cc, Cf, D1, db1, D2, db2,
           Wf, bf, Wr, br, Wb, bb, Wk, bk):
    b1r, b2r, b3r = b1[None, :], b2[None, :], b3[None, :]
    db1r, db2r = db1[None, :], db2[None, :]
    bfr, brr, bbr, bkr = bf[None, :], br[None, :], bb[None, :], bk[None, :]

    out_shapes = (
        jax.ShapeDtypeStruct((B, FEAT), jnp.float32),
        jax.ShapeDtypeStruct((B, ROLES), jnp.float32),
        jax.ShapeDtypeStruct((B, 2), jnp.float32),
        jax.ShapeDtypeStruct((B, 2), jnp.float32),
        jax.ShapeDtypeStruct((B, DM), jnp.float32),
        jax.ShapeDtypeStruct((B, 1), jnp.int32),
        jax.ShapeDtypeStruct((B, 1), jnp.int32),
        jax.ShapeDtypeStruct((1, 1), jnp.float32),
        jax.ShapeDtypeStruct((1, 1), jnp.float32),
        jax.ShapeDtypeStruct((1, 1), jnp.float32),
        jax.ShapeDtypeStruct((1, 1), jnp.float32),
        jax.ShapeDtypeStruct((1, 1), jnp.int32),
        jax.ShapeDtypeStruct((1, 1), jnp.int32),
    )
    in_specs = [
        _batched(FEAT),
        _full((FEAT, HID)), _full((1, HID)),
        _full((HID, HID)), _full((1, HID)),
        _full((HID, DM)), _full((1, DM)),
        _full((NC, DM)), _full((NF, DM)),
        _full((DM, HID)), _full((1, HID)),
        _full((HID, HID)), _full((1, HID)),
        _full((HID, FEAT)), _full((1, FEAT)),
        _full((HID, ROLES)), _full((1, ROLES)),
        _full((HID, 2)), _full((1, 2)),
        _full((HID, 2)), _full((1, 2)),
    ]
    out_specs = (
        _batched(FEAT), _batched(ROLES), _batched(2), _batched(2),
        _batched(DM), _batched(1), _batched(1),
        _full((1, 1)), _full((1, 1)), _full((1, 1)), _full((1, 1)),
        _full((1, 1)), _full((1, 1)),
    )
    scratch = [
        pltpu.VMEM((1, NC), jnp.float32),
        pltpu.VMEM((1, NF), jnp.float32),
        pltpu.VMEM((1, 2), jnp.float32),
    ]

    outs = pl.pallas_call(
        _vq_kernel,
        grid=(NSTEPS,),
        in_specs=in_specs,
        out_specs=out_specs,
        out_shape=out_shapes,
        scratch_shapes=scratch,
    )(feat, W1, b1r, W2, b2r, W3, b3r, Cc, Cf, D1, db1r, D2, db2r,
      Wf, bfr, Wr, brr, Wb, bbr, Wk, bkr)

    (feat_out, role_logits, bounce_logits, break_logits, z, cidx, fidx,
     commit_c, commit_f, ent_c, ent_f, used_c, used_f) = outs

    return (feat_out, role_logits, bounce_logits, break_logits, z,
            cidx[:, 0], fidx[:, 0],
            commit_c[0, 0], commit_f[0, 0], ent_c[0, 0], ent_f[0, 0],
            used_c[0, 0], used_f[0, 0])


# speedup vs baseline: 1.7777x; 1.0242x over previous
import functools

import jax
import jax.numpy as jnp
from jax import lax
from jax.experimental import pallas as pl
from jax.experimental.pallas import tpu as pltpu

FEAT = 256
HID = 128
DM = 64
NC = 256
NF = 1024
ROLES = 8
B = 32768
BETA = 0.25

BB = 2048
NSTEPS = B // BB
NHEAD = ROLES + 2 + 2


_SQRT_HALF = 0.7071067811865476


def _gelu(x):
    return 0.5 * x * (1.0 + lax.erf(x * _SQRT_HALF))


def _nearest_onehot(x, cb):
    e2 = jnp.sum(cb * cb, axis=1)[None, :]
    xe = lax.dot_general(x, cb, (((1,), (1,)), ((), ())),
                         preferred_element_type=jnp.float32)
    score = xe - 0.5 * e2
    maxv = jnp.max(score, axis=1, keepdims=True)
    iota = lax.broadcasted_iota(jnp.int32, score.shape, 1)
    k = cb.shape[0]
    idx = jnp.min(jnp.where(score == maxv, iota, k), axis=1, keepdims=True)
    onehot = (iota == idx).astype(jnp.float32)
    return idx, onehot


def _vq_kernel(
    feat_ref, W1_ref, b1_ref, W2_ref, b2_ref, W3_ref, b3_ref,
    Cc_ref, Cf_ref, D1_ref, db1_ref, D2_ref, db2_ref,
    Wf_ref, bf_ref, Wh_ref, bh_ref,
    feat_out_ref, heads_ref, z_ref, cidx_ref, fidx_ref,
    commit_c_ref, commit_f_ref, ent_c_ref, ent_f_ref, used_c_ref, used_f_ref,
    counts_c_acc, counts_f_acc, sse_acc,
):
    step = pl.program_id(0)

    @pl.when(step == 0)
    def _init():
        counts_c_acc[...] = jnp.zeros_like(counts_c_acc)
        counts_f_acc[...] = jnp.zeros_like(counts_f_acc)
        sse_acc[...] = jnp.zeros_like(sse_acc)

    dot = functools.partial(jnp.dot, preferred_element_type=jnp.float32)

    f = feat_ref[...]
    h = _gelu(dot(f, W1_ref[...]) + b1_ref[...])
    h = _gelu(dot(h, W2_ref[...]) + b2_ref[...])
    z = dot(h, W3_ref[...]) + b3_ref[...]
    z_ref[...] = z

    Cc = Cc_ref[...]
    cidx, onehot_c = _nearest_onehot(z, Cc)
    cq = dot(onehot_c, Cc)
    cidx_ref[...] = cidx
    ones_row = jnp.ones((1, BB), jnp.float32)
    counts_c_acc[...] += dot(ones_row, onehot_c)
    sse_acc[:, 0:1] += jnp.sum((z - cq) ** 2).reshape(1, 1)

    res = z - cq
    Cf = Cf_ref[...]
    fidx, onehot_f = _nearest_onehot(res, Cf)
    fq = dot(onehot_f, Cf)
    fidx_ref[...] = fidx
    counts_f_acc[...] += dot(ones_row, onehot_f)
    sse_acc[:, 1:2] += jnp.sum((res - fq) ** 2).reshape(1, 1)

    dec = cq + fq
    t = _gelu(dot(dec, D1_ref[...]) + db1_ref[...])
    t = _gelu(dot(t, D2_ref[...]) + db2_ref[...])
    feat_out_ref[...] = dot(t, Wf_ref[...]) + bf_ref[...]
    heads_ref[...] = dot(t, Wh_ref[...]) + bh_ref[...]

    @pl.when(step == NSTEPS - 1)
    def _finalize():
        inv = 1.0 / (B * DM)
        sse = sse_acc[...]
        commit_c_ref[...] = BETA * inv * sse[:, 0:1]
        commit_f_ref[...] = BETA * inv * sse[:, 1:2]
        cc = counts_c_acc[...]
        cf = counts_f_acc[...]
        uc = cc * (1.0 / B) + 1e-10
        uf = cf * (1.0 / B) + 1e-10
        ent_c_ref[...] = -jnp.sum(uc * jnp.log(uc)).reshape(1, 1)
        ent_f_ref[...] = -jnp.sum(uf * jnp.log(uf)).reshape(1, 1)
        used_c_ref[...] = jnp.sum((cc > 0).astype(jnp.int32)).reshape(1, 1)
        used_f_ref[...] = jnp.sum((cf > 0).astype(jnp.int32)).reshape(1, 1)


def _full(shape):
    nd = len(shape)
    return pl.BlockSpec(shape, lambda i: (0,) * nd)


def _batched(cols):
    return pl.BlockSpec((BB, cols), lambda i: (i, 0))


@jax.jit
def kernel(feat, W1, b1, W2, b2, W3, b3, Cc, Cf, D1, db1, D2, db2,
           Wf, bf, Wr, br, Wb, bb, Wk, bk):
    b1r, b2r, b3r = b1[None, :], b2[None, :], b3[None, :]
    db1r, db2r = db1[None, :], db2[None, :]
    bfr = bf[None, :]
    Wh = jnp.concatenate([Wr, Wb, Wk], axis=1)
    bh = jnp.concatenate([br, bb, bk])[None, :]

    out_shapes = (
        jax.ShapeDtypeStruct((B, FEAT), jnp.float32),
        jax.ShapeDtypeStruct((B, NHEAD), jnp.float32),
        jax.ShapeDtypeStruct((B, DM), jnp.float32),
        jax.ShapeDtypeStruct((B, 1), jnp.int32),
        jax.ShapeDtypeStruct((B, 1), jnp.int32),
        jax.ShapeDtypeStruct((1, 1), jnp.float32),
        jax.ShapeDtypeStruct((1, 1), jnp.float32),
        jax.ShapeDtypeStruct((1, 1), jnp.float32),
        jax.ShapeDtypeStruct((1, 1), jnp.float32),
        jax.ShapeDtypeStruct((1, 1), jnp.int32),
        jax.ShapeDtypeStruct((1, 1), jnp.int32),
    )
    in_specs = [
        _batched(FEAT),
        _full((FEAT, HID)), _full((1, HID)),
        _full((HID, HID)), _full((1, HID)),
        _full((HID, DM)), _full((1, DM)),
        _full((NC, DM)), _full((NF, DM)),
        _full((DM, HID)), _full((1, HID)),
        _full((HID, HID)), _full((1, HID)),
        _full((HID, FEAT)), _full((1, FEAT)),
        _full((HID, NHEAD)), _full((1, NHEAD)),
    ]
    out_specs = (
        _batched(FEAT), _batched(NHEAD),
        _batched(DM), _batched(1), _batched(1),
        _full((1, 1)), _full((1, 1)), _full((1, 1)), _full((1, 1)),
        _full((1, 1)), _full((1, 1)),
    )
    scratch = [
        pltpu.VMEM((1, NC), jnp.float32),
        pltpu.VMEM((1, NF), jnp.float32),
        pltpu.VMEM((1, 2), jnp.float32),
    ]

    outs = pl.pallas_call(
        _vq_kernel,
        grid=(NSTEPS,),
        in_specs=in_specs,
        out_specs=out_specs,
        out_shape=out_shapes,
        scratch_shapes=scratch,
    )(feat, W1, b1r, W2, b2r, W3, b3r, Cc, Cf, D1, db1r, D2, db2r,
      Wf, bfr, Wh, bh)

    (feat_out, heads, z, cidx, fidx,
     commit_c, commit_f, ent_c, ent_f, used_c, used_f) = outs

    return (feat_out, heads[:, 0:ROLES], heads[:, ROLES:ROLES + 2],
            heads[:, ROLES + 2:ROLES + 4], z,
            cidx[:, 0], fidx[:, 0],
            commit_c[0, 0], commit_f[0, 0], ent_c[0, 0], ent_f[0, 0],
            used_c[0, 0], used_f[0, 0])
